# Initial kernel scaffold; baseline (speedup 1.0000x reference)
import jax
import jax.numpy as jnp
from jax.experimental import pallas as pl


def kernel(x, edge_index, batch, W1, b1, W2, b2, W3, b3):
    raise NotImplementedError("write your pallas kernel here")


# speedup vs baseline: 1.8609x; 1.8609x over previous
import functools

import jax
import jax.numpy as jnp
from jax import lax
from jax.experimental import pallas as pl
from jax.experimental.pallas import tpu as pltpu
from jax.experimental.pallas import tpu_sc as plsc

N = 10000
E = 320000
D = 128
G = 64

NC = 2
NS = 16
CK = 128
NCH = 160
EPS = NCH * CK
E_PAD = NS * EPS
BLK = 8
NBLK = NCH // BLK
HALF = 5120
ACC_ROWS = 5376
RPS = ACC_ROWS // NS
N_PAD = NC * HALF


def _mesh():
    return plsc.VectorSubcoreMesh(core_axis_name="c", subcore_axis_name="s")


def _sc_aggregate(h, src3, dst4):
    @functools.partial(
        pl.kernel,
        out_type=jax.ShapeDtypeStruct((N_PAD, D), jnp.float32),
        mesh=_mesh(),
        scratch_types=[
            pltpu.VMEM((NCH, CK), jnp.int32),
            pltpu.VMEM((2, BLK, CK), jnp.int32),
            pltpu.VMEM((CK,), jnp.int32),
            pltpu.VMEM((CK,), jnp.int32),
            pltpu.VMEM((CK, D), jnp.float32),
            pltpu.VMEM((CK, D), jnp.float32),
            pltpu.VMEM((16, D), jnp.float32),
            pltpu.VMEM_SHARED((ACC_ROWS, D), jnp.float32),
            pltpu.SemaphoreType.DMA,
            pltpu.SemaphoreType.DMA,
            pltpu.SemaphoreType.DMA,
        ],
    )
    def k(h_hbm, src_hbm, dst_hbm, out_hbm,
          src_v, dstb, dloc0, dloc1, rows0, rows1, zbuf, acc,
          sem0, sem1, semi):
        c = lax.axis_index("c")
        s = lax.axis_index("s")
        lo = c * HALF
        hi = lo + HALF

        @pl.loop(0, 16)
        def _(r):
            @pl.loop(0, D, step=16)
            def _(col):
                zbuf[r, pl.ds(col, 16)] = jnp.zeros((16,), jnp.float32)

        @pl.loop(0, RPS // 16)
        def _(t):
            pltpu.sync_copy(zbuf, acc.at[pl.ds(s * RPS + t * 16, 16)])

        pltpu.sync_copy(src_hbm.at[s], src_v)

        plsc.subcore_barrier()

        def remap(dstb_p, t, dloc):
            @pl.loop(0, CK, step=16)
            def _(col):
                d = dstb_p[t, pl.ds(col, 16)]
                inr = jnp.logical_and(d >= lo, d < hi)
                dloc[pl.ds(col, 16)] = jnp.where(inr, d - lo, HALF)

        def chunk_pair(j, dstb_p, t):
            pltpu.async_copy(h_hbm.at[src_v.at[j + 1]], rows1, sem1)
            remap(dstb_p, t, dloc0)
            pltpu.make_async_copy(h_hbm.at[src_v.at[j]], rows0, sem0).wait()
            pltpu.sync_copy(rows0, acc.at[dloc0], add=True)

            @pl.when(j + 2 < NCH)
            def _():
                pltpu.async_copy(h_hbm.at[src_v.at[j + 2]], rows0, sem0)

            remap(dstb_p, t + 1, dloc1)
            pltpu.make_async_copy(h_hbm.at[src_v.at[j + 1]], rows1, sem1).wait()
            pltpu.sync_copy(rows1, acc.at[dloc1], add=True)

        pltpu.sync_copy(dst_hbm.at[s, 0], dstb.at[0])
        pltpu.async_copy(h_hbm.at[src_v.at[0]], rows0, sem0)

        @pl.loop(0, NBLK, step=2)
        def _(b):
            pltpu.async_copy(dst_hbm.at[s, b + 1], dstb.at[1], semi)

            @pl.loop(0, BLK, step=2)
            def _(t):
                chunk_pair(b * BLK + t, dstb.at[0], t)

            pltpu.make_async_copy(dst_hbm.at[s, b + 1], dstb.at[1], semi).wait()

            @pl.when(b + 2 < NBLK)
            def _():
                pltpu.async_copy(dst_hbm.at[s, b + 2], dstb.at[0], semi)

            @pl.loop(0, BLK, step=2)
            def _(t):
                chunk_pair((b + 1) * BLK + t, dstb.at[1], t)

            @pl.when(b + 2 < NBLK)
            def _():
                pltpu.make_async_copy(dst_hbm.at[s, b + 2], dstb.at[0], semi).wait()

        plsc.subcore_barrier()

        pltpu.sync_copy(
            acc.at[pl.ds(s * (HALF // NS), HALF // NS)],
            out_hbm.at[pl.ds(c * HALF + s * (HALF // NS), HALF // NS)],
        )

    return k(h, src3, dst4)


def _tc_dense(agg, w_t, b8, relu):
    blk = N_PAD // 16

    def body(a_ref, w_ref, b_ref, o_ref):
        y = jnp.dot(a_ref[...], w_ref[...], preferred_element_type=jnp.float32)
        y = y + b_ref[0:1, :]
        if relu:
            y = jnp.where(y >= 0, y, 0.01 * y)
        nrm = jnp.sqrt(jnp.sum(y * y, axis=1, keepdims=True))
        o_ref[...] = y / jnp.maximum(nrm, 1e-12)

    return pl.pallas_call(
        body,
        grid=(16,),
        in_specs=[
            pl.BlockSpec((blk, D), lambda i: (i, 0)),
            pl.BlockSpec((D, D), lambda i: (0, 0)),
            pl.BlockSpec((8, D), lambda i: (0, 0)),
        ],
        out_specs=pl.BlockSpec((blk, D), lambda i: (i, 0)),
        out_shape=jax.ShapeDtypeStruct((N_PAD, D), jnp.float32),
    )(agg, w_t, b8)


def _sc_gather_rows(h, idx):
    @functools.partial(
        pl.kernel,
        out_type=jax.ShapeDtypeStruct((G, D), jnp.float32),
        mesh=_mesh(),
        scratch_types=[
            pltpu.VMEM((G,), jnp.int32),
            pltpu.VMEM((G, D), jnp.float32),
            pltpu.SemaphoreType.DMA,
        ],
    )
    def k(h_hbm, idx_hbm, out_hbm, idx_v, rows_v, sem):
        c = lax.axis_index("c")
        s = lax.axis_index("s")

        @pl.when(jnp.logical_and(c == 0, s == 0))
        def _():
            pltpu.sync_copy(idx_hbm, idx_v)
            pltpu.async_copy(h_hbm.at[idx_v], rows_v, sem).wait()
            pltpu.sync_copy(rows_v, out_hbm)

    return k(h, idx)


def kernel(x, edge_index, batch, W1, b1, W2, b2, W3, b3):
    pad = E_PAD - E
    src3 = jnp.concatenate(
        [edge_index[0], jnp.zeros((pad,), jnp.int32)]).reshape(NS, NCH, CK)
    dst4 = jnp.concatenate(
        [edge_index[1], jnp.full((pad,), N, jnp.int32)]).reshape(
            NS, NBLK, BLK, CK)

    w1t, w2t, w3t = W1.T, W2.T, W3.T
    b1_8 = jnp.broadcast_to(b1[None, :], (8, D))
    b2_8 = jnp.broadcast_to(b2[None, :], (8, D))
    b3_8 = jnp.broadcast_to(b3[None, :], (8, D))

    a1 = _sc_aggregate(x, src3, dst4)
    h1 = _tc_dense(a1, w1t, b1_8, relu=True)
    a2 = _sc_aggregate(h1, src3, dst4)
    h2 = _tc_dense(a2, w2t, b2_8, relu=True)
    a3 = _sc_aggregate(h2, src3, dst4)
    h3 = _tc_dense(a3, w3t, b3_8, relu=False)

    head = jnp.ones((1,), dtype=bool)
    changed = batch[1:] != batch[:-1]
    mask = jnp.concatenate([head, changed])
    idx = jnp.nonzero(mask, size=G, fill_value=0)[0].astype(jnp.int32)

    return _sc_gather_rows(h3, idx)


# speedup vs baseline: 6.6203x; 3.5575x over previous
import functools

import jax
import jax.numpy as jnp
from jax import lax
from jax.experimental import pallas as pl
from jax.experimental.pallas import tpu as pltpu
from jax.experimental.pallas import tpu_sc as plsc

N = 10000
E = 320000
D = 128
G = 64

NC = 2
NS = 16
NW = NC * NS
CK = 128
HALF = 5120
TRASH = HALF
ACC_ROWS = 5376
RPS = ACC_ROWS // NS
N_PAD = NC * HALF

EPR = E // NS
SEG = 4000
NSEG = EPR // SEG
SEGC = SEG + CK
SLOT = 24576
NCHMAX = 158
GRP = 16
GMAX = 10


def _mesh():
    return plsc.VectorSubcoreMesh(core_axis_name="c", subcore_axis_name="s")


_SC_PARAMS = pltpu.CompilerParams(needs_layout_passes=False)


def _sc_partition(src_flat, dst_flat):
    @functools.partial(
        pl.kernel,
        out_type=(
            jax.ShapeDtypeStruct((NW * SLOT,), jnp.int32),
            jax.ShapeDtypeStruct((NW * SLOT,), jnp.int32),
            jax.ShapeDtypeStruct((NW * 16,), jnp.int32),
        ),
        mesh=_mesh(),
        scratch_types=[
            pltpu.VMEM((SEG,), jnp.int32),
            pltpu.VMEM((SEG,), jnp.int32),
            pltpu.VMEM((SEGC,), jnp.int32),
            pltpu.VMEM((SEGC,), jnp.int32),
            pltpu.VMEM((16,), jnp.int32),
        ],
        compiler_params=_SC_PARAMS,
    )
    def k(src_hbm, dst_hbm, srcp_hbm, dlocp_hbm, cnt_hbm,
          rawsrc, rawdst, csrc, cdst, cntv):
        c = lax.axis_index("c")
        s = lax.axis_index("s")
        w = c * NS + s
        lo = c * HALF
        hi = lo + HALF
        wslot = w * SLOT

        def seg_body(g, off):
            base = s * EPR + g * SEG
            pltpu.sync_copy(src_hbm.at[pl.ds(base, SEG)], rawsrc)
            pltpu.sync_copy(dst_hbm.at[pl.ds(base, SEG)], rawdst)

            @pl.loop(0, SEGC, step=16)
            def _(i):
                csrc[pl.ds(i, 16)] = jnp.zeros((16,), jnp.int32)
                cdst[pl.ds(i, 16)] = jnp.full((16,), TRASH, jnp.int32)

            def it_body(it, segoff):
                sv = rawsrc[pl.ds(it * 16, 16)]
                dv = rawdst[pl.ds(it * 16, 16)]
                m = jnp.logical_and(dv >= lo, dv < hi)
                mi = m.astype(jnp.int32)
                idx = segoff + plsc.cumsum(mi) - 1
                plsc.store_scatter(csrc, [idx], sv, mask=m)
                plsc.store_scatter(cdst, [idx], dv - lo, mask=m)
                return segoff + jnp.sum(mi)

            segoff = lax.fori_loop(0, SEG // 16, it_body, 0)
            o8 = pl.multiple_of(wslot + off, 8)
            pltpu.sync_copy(csrc, srcp_hbm.at[pl.ds(o8, SEGC)])
            pltpu.sync_copy(cdst, dlocp_hbm.at[pl.ds(o8, SEGC)])
            return off + ((segoff + 7) // 8) * 8

        total = lax.fori_loop(0, NSEG, seg_body, 0)
        cntv[...] = jnp.full((16,), total, jnp.int32)
        pltpu.sync_copy(cntv, cnt_hbm.at[pl.ds(w * 16, 16)])

    return k(src_flat, dst_flat)


def _sc_aggregate(h, src4, dloc4, counts2):
    @functools.partial(
        pl.kernel,
        out_type=jax.ShapeDtypeStruct((N_PAD, D), jnp.float32),
        mesh=_mesh(),
        scratch_types=[
            pltpu.VMEM((160, CK), jnp.int32),
            pltpu.VMEM((2, GRP, CK), jnp.int32),
            pltpu.VMEM((16,), jnp.int32),
            pltpu.VMEM((CK, D), jnp.float32),
            pltpu.VMEM((CK, D), jnp.float32),
            pltpu.VMEM((16, D), jnp.float32),
            pltpu.VMEM_SHARED((ACC_ROWS, D), jnp.float32),
            pltpu.SemaphoreType.DMA,
            pltpu.SemaphoreType.DMA,
            pltpu.SemaphoreType.DMA,
            pltpu.SemaphoreType.DMA,
        ],
    )
    def k(h_hbm, src_hbm, dloc_hbm, cnt_hbm, out_hbm,
          src_v, dlg, cntv, rows0, rows1, zbuf, acc, sem0, sem1, sems, semg):
        c = lax.axis_index("c")
        s = lax.axis_index("s")
        w = c * NS + s

        @pl.loop(0, 16)
        def _(r):
            @pl.loop(0, D, step=16)
            def _(col):
                zbuf[r, pl.ds(col, 16)] = jnp.zeros((16,), jnp.float32)

        @pl.loop(0, RPS // 16)
        def _(t):
            pltpu.sync_copy(zbuf, acc.at[pl.ds(s * RPS + t * 16, 16)])

        pltpu.sync_copy(cnt_hbm.at[w], cntv)
        cnt = cntv[pl.ds(0, 16)][0]
        nch = (cnt + CK - 1) // CK

        @pl.loop(0, 20)
        def _(blk):
            pltpu.async_copy(src_hbm.at[w, blk], src_v.at[pl.ds(blk * 8, 8)],
                             sems)

        @pl.loop(0, 20)
        def _(blk):
            pltpu.make_async_copy(src_hbm.at[w, blk],
                                  src_v.at[pl.ds(blk * 8, 8)], sems).wait()

        plsc.subcore_barrier()

        def gather(j, rows, sem):
            pltpu.async_copy(h_hbm.at[src_v.at[j]], rows, sem)

        def wait_gather(j, rows, sem):
            pltpu.make_async_copy(h_hbm.at[src_v.at[j]], rows, sem).wait()

        def group_pair(g, p, q):
            @pl.when(g == 0)
            def _():
                pltpu.sync_copy(dloc_hbm.at[w, 0], dlg.at[0])

                @pl.when(0 < nch)
                def _():
                    gather(0, rows0, sem0)

            @pl.when((g + 1) * GRP < nch)
            def _():
                pltpu.async_copy(dloc_hbm.at[w, g + 1], dlg.at[q], semg)

            def chunks_of(g2, pp):
                @pl.loop(0, GRP, step=2)
                def _(t):
                    j = g2 * GRP + t

                    @pl.when(j + 1 < nch)
                    def _():
                        gather(j + 1, rows1, sem1)

                    @pl.when(j < nch)
                    def _():
                        wait_gather(j, rows0, sem0)
                        pltpu.sync_copy(rows0, acc.at[dlg.at[pp, t]],
                                        add=True)

                    @pl.when(j + 2 < nch)
                    def _():
                        gather(j + 2, rows0, sem0)

                    @pl.when(j + 1 < nch)
                    def _():
                        wait_gather(j + 1, rows1, sem1)
                        pltpu.sync_copy(rows1, acc.at[dlg.at[pp, t + 1]],
                                        add=True)

            chunks_of(g, p)

            @pl.when((g + 1) * GRP < nch)
            def _():
                pltpu.make_async_copy(dloc_hbm.at[w, g + 1], dlg.at[q],
                                      semg).wait()

            @pl.when((g + 2) * GRP < nch)
            def _():
                pltpu.async_copy(dloc_hbm.at[w, g + 2], dlg.at[p], semg)

            chunks_of(g + 1, q)

            @pl.when((g + 2) * GRP < nch)
            def _():
                pltpu.make_async_copy(dloc_hbm.at[w, g + 2], dlg.at[p],
                                      semg).wait()

        @pl.loop(0, GMAX, step=2)
        def _(g):
            group_pair(g, 0, 1)

        plsc.subcore_barrier()

        pltpu.sync_copy(
            acc.at[pl.ds(s * (HALF // NS), HALF // NS)],
            out_hbm.at[pl.ds(c * HALF + s * (HALF // NS), HALF // NS)],
        )

    return k(h, src4, dloc4, counts2)


def _tc_dense(agg, w_t, b8, relu):
    blk = N_PAD // 16

    def body(a_ref, w_ref, b_ref, o_ref):
        y = jnp.dot(a_ref[...], w_ref[...], preferred_element_type=jnp.float32,
                    precision=lax.Precision.HIGHEST)
        y = y + b_ref[0:1, :]
        if relu:
            y = jnp.where(y >= 0, y, 0.01 * y)
        nrm = jnp.sqrt(jnp.sum(y * y, axis=1, keepdims=True))
        o_ref[...] = y / jnp.maximum(nrm, 1e-12)

    return pl.pallas_call(
        body,
        grid=(16,),
        in_specs=[
            pl.BlockSpec((blk, D), lambda i: (i, 0)),
            pl.BlockSpec((D, D), lambda i: (0, 0)),
            pl.BlockSpec((8, D), lambda i: (0, 0)),
        ],
        out_specs=pl.BlockSpec((blk, D), lambda i: (i, 0)),
        out_shape=jax.ShapeDtypeStruct((N_PAD, D), jnp.float32),
    )(agg, w_t, b8)


def _sc_gather_rows(h, idx):
    @functools.partial(
        pl.kernel,
        out_type=jax.ShapeDtypeStruct((G, D), jnp.float32),
        mesh=_mesh(),
        scratch_types=[
            pltpu.VMEM((G,), jnp.int32),
            pltpu.VMEM((G, D), jnp.float32),
            pltpu.SemaphoreType.DMA,
        ],
    )
    def k(h_hbm, idx_hbm, out_hbm, idx_v, rows_v, sem):
        c = lax.axis_index("c")
        s = lax.axis_index("s")

        @pl.when(jnp.logical_and(c == 0, s == 0))
        def _():
            pltpu.sync_copy(idx_hbm, idx_v)
            pltpu.async_copy(h_hbm.at[idx_v], rows_v, sem).wait()
            pltpu.sync_copy(rows_v, out_hbm)

    return k(h, idx)


def kernel(x, edge_index, batch, W1, b1, W2, b2, W3, b3):
    srcp, dlocp, counts = _sc_partition(edge_index[0], edge_index[1])
    src4 = srcp.reshape(NW, 24, 8, CK)
    dloc4 = dlocp.reshape(NW, 12, GRP, CK)
    counts2 = counts.reshape(NW, 16)

    w1t, w2t, w3t = W1.T, W2.T, W3.T
    b1_8 = jnp.broadcast_to(b1[None, :], (8, D))
    b2_8 = jnp.broadcast_to(b2[None, :], (8, D))
    b3_8 = jnp.broadcast_to(b3[None, :], (8, D))

    a1 = _sc_aggregate(x, src4, dloc4, counts2)
    h1 = _tc_dense(a1, w1t, b1_8, relu=True)
    a2 = _sc_aggregate(h1, src4, dloc4, counts2)
    h2 = _tc_dense(a2, w2t, b2_8, relu=True)
    a3 = _sc_aggregate(h2, src4, dloc4, counts2)
    h3 = _tc_dense(a3, w3t, b3_8, relu=False)

    head = jnp.ones((1,), dtype=bool)
    changed = batch[1:] != batch[:-1]
    mask = jnp.concatenate([head, changed])
    idx = jnp.nonzero(mask, size=G, fill_value=0)[0].astype(jnp.int32)

    return _sc_gather_rows(h3, idx)


# speedup vs baseline: 6.9536x; 1.0503x over previous
import functools

import jax
import jax.numpy as jnp
from jax import lax
from jax.experimental import pallas as pl
from jax.experimental.pallas import tpu as pltpu
from jax.experimental.pallas import tpu_sc as plsc

N = 10000
E = 320000
D = 128
G = 64

NC = 2
NS = 16
NW = NC * NS
CK = 128
HALF = 5120
TRASH = HALF
ACC_ROWS = 5376
RPS = ACC_ROWS // NS
N_PAD = NC * HALF

EPR = E // NS
SEG = 4000
NSEG = EPR // SEG
SEGC = SEG + CK
SLOT = 24576
NCHMAX = 158
GRP = 16
GMAX = 10


def _mesh():
    return plsc.VectorSubcoreMesh(core_axis_name="c", subcore_axis_name="s")


_SC_PARAMS = pltpu.CompilerParams(needs_layout_passes=False)


def _sc_partition(src_flat, dst_flat):
    @functools.partial(
        pl.kernel,
        out_type=(
            jax.ShapeDtypeStruct((NW * SLOT,), jnp.int32),
            jax.ShapeDtypeStruct((NW * SLOT,), jnp.int32),
            jax.ShapeDtypeStruct((NW * 16,), jnp.int32),
        ),
        mesh=_mesh(),
        scratch_types=[
            pltpu.VMEM((SEG,), jnp.int32),
            pltpu.VMEM((SEG,), jnp.int32),
            pltpu.VMEM((SEGC,), jnp.int32),
            pltpu.VMEM((SEGC,), jnp.int32),
            pltpu.VMEM((16,), jnp.int32),
        ],
        compiler_params=_SC_PARAMS,
    )
    def k(src_hbm, dst_hbm, srcp_hbm, dlocp_hbm, cnt_hbm,
          rawsrc, rawdst, csrc, cdst, cntv):
        c = lax.axis_index("c")
        s = lax.axis_index("s")
        w = c * NS + s
        lo = c * HALF
        hi = lo + HALF
        wslot = w * SLOT

        def seg_body(g, off):
            base = s * EPR + g * SEG
            pltpu.sync_copy(src_hbm.at[pl.ds(base, SEG)], rawsrc)
            pltpu.sync_copy(dst_hbm.at[pl.ds(base, SEG)], rawdst)

            @pl.loop(0, SEGC, step=16)
            def _(i):
                csrc[pl.ds(i, 16)] = jnp.zeros((16,), jnp.int32)
                cdst[pl.ds(i, 16)] = jnp.full((16,), TRASH, jnp.int32)

            def it_body(it, segoff):
                sv = rawsrc[pl.ds(it * 16, 16)]
                dv = rawdst[pl.ds(it * 16, 16)]
                m = jnp.logical_and(dv >= lo, dv < hi)
                mi = m.astype(jnp.int32)
                idx = segoff + plsc.cumsum(mi) - 1
                plsc.store_scatter(csrc, [idx], sv, mask=m)
                plsc.store_scatter(cdst, [idx], dv - lo, mask=m)
                return segoff + jnp.sum(mi)

            segoff = lax.fori_loop(0, SEG // 16, it_body, 0)
            o8 = pl.multiple_of(wslot + off, 8)
            pltpu.sync_copy(csrc, srcp_hbm.at[pl.ds(o8, SEGC)])
            pltpu.sync_copy(cdst, dlocp_hbm.at[pl.ds(o8, SEGC)])
            return off + ((segoff + 7) // 8) * 8

        total = lax.fori_loop(0, NSEG, seg_body, 0)
        cntv[...] = jnp.full((16,), total, jnp.int32)
        pltpu.sync_copy(cntv, cnt_hbm.at[pl.ds(w * 16, 16)])

    return k(src_flat, dst_flat)


def _sc_aggregate(h, srcg, dlocg, counts2):
    @functools.partial(
        pl.kernel,
        out_type=jax.ShapeDtypeStruct((N_PAD, D), jnp.float32),
        mesh=_mesh(),
        scratch_types=[
            pltpu.VMEM((3, GRP, CK), jnp.int32),
            pltpu.VMEM((3, GRP, CK), jnp.int32),
            pltpu.VMEM((16,), jnp.int32),
            pltpu.VMEM((4, CK, D), jnp.float32),
            pltpu.VMEM((16, D), jnp.float32),
            pltpu.VMEM_SHARED((ACC_ROWS, D), jnp.float32),
            [pltpu.SemaphoreType.DMA] * 4,
            [pltpu.SemaphoreType.DMA] * 4,
            pltpu.SemaphoreType.DMA,
            pltpu.SemaphoreType.DMA,
        ],
    )
    def k(h_hbm, src_hbm, dloc_hbm, cnt_hbm, out_hbm,
          sgrp, dgrp, cntv, rows, zbuf, acc, semg, semc, semz, semd):
        c = lax.axis_index("c")
        s = lax.axis_index("s")
        w = c * NS + s

        @pl.loop(0, 16)
        def _(r):
            @pl.loop(0, D, step=16)
            def _(col):
                zbuf[r, pl.ds(col, 16)] = jnp.zeros((16,), jnp.float32)

        @pl.loop(0, RPS // 16)
        def _(t):
            pltpu.async_copy(zbuf, acc.at[pl.ds(s * RPS + t * 16, 16)], semz)

        pltpu.sync_copy(cnt_hbm.at[w], cntv)
        cnt = cntv[pl.ds(0, 16)][0]
        nch = (cnt + CK - 1) // CK

        @pl.loop(0, RPS // 16)
        def _(t):
            pltpu.make_async_copy(zbuf, acc.at[pl.ds(s * RPS + t * 16, 16)],
                                  semz).wait()

        plsc.subcore_barrier()

        def stage_group(gi, bi):
            pltpu.async_copy(src_hbm.at[w, gi], sgrp.at[bi], semd)
            pltpu.async_copy(dloc_hbm.at[w, gi], dgrp.at[bi], semd)

        def wait_group(gi, bi):
            pltpu.make_async_copy(src_hbm.at[w, gi], sgrp.at[bi], semd).wait()
            pltpu.make_async_copy(dloc_hbm.at[w, gi], dgrp.at[bi],
                                  semd).wait()

        def issue_gather(idxrow, r):
            pltpu.async_copy(h_hbm.at[idxrow], rows.at[r], semg[r])

        def wait_gather(idxrow, r):
            pltpu.make_async_copy(h_hbm.at[idxrow], rows.at[r],
                                  semg[r]).wait()

        def issue_scatter(bi, tk, r):
            pltpu.async_copy(rows.at[r], acc.at[dgrp.at[bi, tk]], semc[r],
                             add=True)

        def wait_scatter(bi, tk, r):
            pltpu.make_async_copy(rows.at[r], acc.at[dgrp.at[bi, tk]],
                                  semc[r]).wait()

        def slot_loop(g2, bi, t, kk):
            tk = t + kk
            j = g2 * GRP + tk
            r = kk
            r2 = (kk + 2) % 4

            @pl.when(jnp.logical_and(tk >= 2, j - 2 < nch))
            def _():
                wait_scatter(bi, tk - 2, r2)

            @pl.when(j + 2 < nch)
            def _():
                issue_gather(sgrp.at[bi, tk + 2], r2)

            @pl.when(j < nch)
            def _():
                wait_gather(sgrp.at[bi, tk], r)
                issue_scatter(bi, tk, r)

        def slot_tail(g2, bi, binext, tk):
            j = g2 * GRP + tk
            r = tk % 4
            r2 = (tk + 2) % 4

            @pl.when(j - 2 < nch)
            def _():
                wait_scatter(bi, tk - 2, r2)

            nxt = (sgrp.at[bi, tk + 2] if tk < 14
                   else sgrp.at[binext, tk - 14])

            @pl.when(j + 2 < nch)
            def _():
                issue_gather(nxt, r2)

            @pl.when(j < nch)
            def _():
                wait_gather(sgrp.at[bi, tk], r)
                issue_scatter(bi, tk, r)

        def chunks_of(g2, bi, binext):
            @pl.loop(0, 12, step=4)
            def _(t):
                slot_loop(g2, bi, t, 0)
                slot_loop(g2, bi, t, 1)
                slot_loop(g2, bi, t, 2)
                slot_loop(g2, bi, t, 3)

            for tk in (12, 13, 14, 15):
                slot_tail(g2, bi, binext, tk)

            @pl.when(g2 * GRP + 14 < nch)
            def _():
                wait_scatter(bi, 14, 2)

            @pl.when(g2 * GRP + 15 < nch)
            def _():
                wait_scatter(bi, 15, 3)

        @pl.loop(0, 12, step=3)
        def _(g):
            @pl.when(g == 0)
            def _():
                pltpu.sync_copy(src_hbm.at[w, 0], sgrp.at[0])
                pltpu.sync_copy(dloc_hbm.at[w, 0], dgrp.at[0])

                @pl.when(GRP < nch)
                def _():
                    pltpu.sync_copy(src_hbm.at[w, 1], sgrp.at[1])
                    pltpu.sync_copy(dloc_hbm.at[w, 1], dgrp.at[1])

                @pl.when(0 < nch)
                def _():
                    issue_gather(sgrp.at[0, 0], 0)

                @pl.when(1 < nch)
                def _():
                    issue_gather(sgrp.at[0, 1], 1)

            @pl.when((g + 2) * GRP < nch)
            def _():
                stage_group(g + 2, 2)

            chunks_of(g, 0, 1)

            @pl.when((g + 2) * GRP < nch)
            def _():
                wait_group(g + 2, 2)

            @pl.when((g + 3) * GRP < nch)
            def _():
                stage_group(g + 3, 0)

            chunks_of(g + 1, 1, 2)

            @pl.when((g + 3) * GRP < nch)
            def _():
                wait_group(g + 3, 0)

            @pl.when((g + 4) * GRP < nch)
            def _():
                stage_group(g + 4, 1)

            chunks_of(g + 2, 2, 0)

            @pl.when((g + 4) * GRP < nch)
            def _():
                wait_group(g + 4, 1)

        plsc.subcore_barrier()

        pltpu.sync_copy(
            acc.at[pl.ds(s * (HALF // NS), HALF // NS)],
            out_hbm.at[pl.ds(c * HALF + s * (HALF // NS), HALF // NS)],
        )

    return k(h, srcg, dlocg, counts2)


def _tc_dense(agg, w_t, b8, relu):
    blk = N_PAD // 16

    def body(a_ref, w_ref, b_ref, o_ref):
        y = jnp.dot(a_ref[...], w_ref[...], preferred_element_type=jnp.float32,
                    precision=lax.Precision.HIGHEST)
        y = y + b_ref[0:1, :]
        if relu:
            y = jnp.where(y >= 0, y, 0.01 * y)
        nrm = jnp.sqrt(jnp.sum(y * y, axis=1, keepdims=True))
        o_ref[...] = y / jnp.maximum(nrm, 1e-12)

    return pl.pallas_call(
        body,
        grid=(16,),
        in_specs=[
            pl.BlockSpec((blk, D), lambda i: (i, 0)),
            pl.BlockSpec((D, D), lambda i: (0, 0)),
            pl.BlockSpec((8, D), lambda i: (0, 0)),
        ],
        out_specs=pl.BlockSpec((blk, D), lambda i: (i, 0)),
        out_shape=jax.ShapeDtypeStruct((N_PAD, D), jnp.float32),
    )(agg, w_t, b8)


def _sc_gather_rows(h, idx):
    @functools.partial(
        pl.kernel,
        out_type=jax.ShapeDtypeStruct((G, D), jnp.float32),
        mesh=_mesh(),
        scratch_types=[
            pltpu.VMEM((G,), jnp.int32),
            pltpu.VMEM((G, D), jnp.float32),
            pltpu.SemaphoreType.DMA,
        ],
    )
    def k(h_hbm, idx_hbm, out_hbm, idx_v, rows_v, sem):
        c = lax.axis_index("c")
        s = lax.axis_index("s")

        @pl.when(jnp.logical_and(c == 0, s == 0))
        def _():
            pltpu.sync_copy(idx_hbm, idx_v)
            pltpu.async_copy(h_hbm.at[idx_v], rows_v, sem).wait()
            pltpu.sync_copy(rows_v, out_hbm)

    return k(h, idx)


def kernel(x, edge_index, batch, W1, b1, W2, b2, W3, b3):
    srcp, dlocp, counts = _sc_partition(edge_index[0], edge_index[1])
    src4 = srcp.reshape(NW, 12, GRP, CK)
    dloc4 = dlocp.reshape(NW, 12, GRP, CK)
    counts2 = counts.reshape(NW, 16)

    w1t, w2t, w3t = W1.T, W2.T, W3.T
    b1_8 = jnp.broadcast_to(b1[None, :], (8, D))
    b2_8 = jnp.broadcast_to(b2[None, :], (8, D))
    b3_8 = jnp.broadcast_to(b3[None, :], (8, D))

    a1 = _sc_aggregate(x, src4, dloc4, counts2)
    h1 = _tc_dense(a1, w1t, b1_8, relu=True)
    a2 = _sc_aggregate(h1, src4, dloc4, counts2)
    h2 = _tc_dense(a2, w2t, b2_8, relu=True)
    a3 = _sc_aggregate(h2, src4, dloc4, counts2)
    h3 = _tc_dense(a3, w3t, b3_8, relu=False)

    head = jnp.ones((1,), dtype=bool)
    changed = batch[1:] != batch[:-1]
    mask = jnp.concatenate([head, changed])
    idx = jnp.nonzero(mask, size=G, fill_value=0)[0].astype(jnp.int32)

    return _sc_gather_rows(h3, idx)


# speedup vs baseline: 7.6765x; 1.1040x over previous
import functools

import jax
import jax.numpy as jnp
from jax import lax
from jax.experimental import pallas as pl
from jax.experimental.pallas import tpu as pltpu
from jax.experimental.pallas import tpu_sc as plsc

N = 10000
E = 320000
D = 128
G = 64

NC = 2
NS = 16
NW = NC * NS
CK = 128
HALF = 5120
TRASH = HALF
ACC_ROWS = 5376
RPS = ACC_ROWS // NS
N_PAD = NC * HALF

EPR = E // NS
SEG = 4000
NSEG = EPR // SEG
SEGC = SEG + CK
SLOT = 24576
NCHMAX = 158
GRP = 16
GMAX = 10


def _mesh():
    return plsc.VectorSubcoreMesh(core_axis_name="c", subcore_axis_name="s")


_SC_PARAMS = pltpu.CompilerParams(needs_layout_passes=False)


def _sc_partition(src_flat, dst_flat, ishead):
    @functools.partial(
        pl.kernel,
        out_type=(
            jax.ShapeDtypeStruct((NW * SLOT,), jnp.int32),
            jax.ShapeDtypeStruct((NW * SLOT,), jnp.int32),
            jax.ShapeDtypeStruct((NW * SLOT,), jnp.int32),
            jax.ShapeDtypeStruct((NW * SLOT,), jnp.int32),
            jax.ShapeDtypeStruct((NW * 16,), jnp.int32),
        ),
        mesh=_mesh(),
        scratch_types=[
            pltpu.VMEM((SEG,), jnp.int32),
            pltpu.VMEM((SEG,), jnp.int32),
            pltpu.VMEM((SEGC,), jnp.int32),
            pltpu.VMEM((SEGC,), jnp.int32),
            pltpu.VMEM((SEGC,), jnp.int32),
            pltpu.VMEM((SEGC,), jnp.int32),
            pltpu.VMEM((N_PAD,), jnp.int32),
            pltpu.VMEM((16,), jnp.int32),
        ],
        compiler_params=_SC_PARAMS,
    )
    def k(src_hbm, dst_hbm, ishead_hbm, srcp_hbm, dlocp_hbm, srcp3_hbm,
          dlocp3_hbm, cnt_hbm,
          rawsrc, rawdst, csrc, cdst, csrc3, cdst3, ishead_v, cntv):
        c = lax.axis_index("c")
        s = lax.axis_index("s")
        w = c * NS + s
        lo = c * HALF
        hi = lo + HALF
        wslot = w * SLOT

        pltpu.sync_copy(ishead_hbm, ishead_v)

        def seg_body(g, carry):
            off, off3 = carry
            base = s * EPR + g * SEG
            pltpu.sync_copy(src_hbm.at[pl.ds(base, SEG)], rawsrc)
            pltpu.sync_copy(dst_hbm.at[pl.ds(base, SEG)], rawdst)

            @pl.loop(0, SEGC, step=16)
            def _(i):
                csrc[pl.ds(i, 16)] = jnp.zeros((16,), jnp.int32)
                cdst[pl.ds(i, 16)] = jnp.full((16,), TRASH, jnp.int32)
                csrc3[pl.ds(i, 16)] = jnp.zeros((16,), jnp.int32)
                cdst3[pl.ds(i, 16)] = jnp.full((16,), TRASH, jnp.int32)

            def it_body(it, soff):
                segoff, segoff3 = soff
                sv = rawsrc[pl.ds(it * 16, 16)]
                dv = rawdst[pl.ds(it * 16, 16)]
                m = jnp.logical_and(dv >= lo, dv < hi)
                mi = m.astype(jnp.int32)
                idx = segoff + plsc.cumsum(mi) - 1
                plsc.store_scatter(csrc, [idx], sv, mask=m)
                plsc.store_scatter(cdst, [idx], dv - lo, mask=m)
                hflag = plsc.load_gather(ishead_v, [dv])
                m3 = jnp.logical_and(m, hflag > 0)
                mi3 = m3.astype(jnp.int32)
                idx3 = segoff3 + plsc.cumsum(mi3) - 1
                plsc.store_scatter(csrc3, [idx3], sv, mask=m3)
                plsc.store_scatter(cdst3, [idx3], dv - lo, mask=m3)
                return segoff + jnp.sum(mi), segoff3 + jnp.sum(mi3)

            segoff, segoff3 = lax.fori_loop(0, SEG // 16, it_body, (0, 0))
            o8 = pl.multiple_of(wslot + off, 8)
            pltpu.sync_copy(csrc, srcp_hbm.at[pl.ds(o8, SEGC)])
            pltpu.sync_copy(cdst, dlocp_hbm.at[pl.ds(o8, SEGC)])
            o83 = pl.multiple_of(wslot + off3, 8)
            pltpu.sync_copy(csrc3, srcp3_hbm.at[pl.ds(o83, SEGC)])
            pltpu.sync_copy(cdst3, dlocp3_hbm.at[pl.ds(o83, SEGC)])
            return (off + ((segoff + 7) // 8) * 8,
                    off3 + ((segoff3 + 7) // 8) * 8)

        total, total3 = lax.fori_loop(0, NSEG, seg_body, (0, 0))
        lanes = lax.iota(jnp.int32, 16)
        cntv[...] = jnp.where(lanes >= 1, total3, total)
        pltpu.sync_copy(cntv, cnt_hbm.at[pl.ds(w * 16, 16)])

    return k(src_flat, dst_flat, ishead)


def _sc_aggregate(h, srcg, dlocg, counts2, lane=0):
    @functools.partial(
        pl.kernel,
        out_type=jax.ShapeDtypeStruct((N_PAD, D), jnp.float32),
        mesh=_mesh(),
        scratch_types=[
            pltpu.VMEM((3, GRP, CK), jnp.int32),
            pltpu.VMEM((3, GRP, CK), jnp.int32),
            pltpu.VMEM((16,), jnp.int32),
            pltpu.VMEM((4, CK, D), jnp.float32),
            pltpu.VMEM((16, D), jnp.float32),
            pltpu.VMEM_SHARED((ACC_ROWS, D), jnp.float32),
            [pltpu.SemaphoreType.DMA] * 4,
            [pltpu.SemaphoreType.DMA] * 4,
            pltpu.SemaphoreType.DMA,
            pltpu.SemaphoreType.DMA,
        ],
    )
    def k(h_hbm, src_hbm, dloc_hbm, cnt_hbm, out_hbm,
          sgrp, dgrp, cntv, rows, zbuf, acc, semg, semc, semz, semd):
        c = lax.axis_index("c")
        s = lax.axis_index("s")
        w = c * NS + s

        @pl.loop(0, 16)
        def _(r):
            @pl.loop(0, D, step=16)
            def _(col):
                zbuf[r, pl.ds(col, 16)] = jnp.zeros((16,), jnp.float32)

        @pl.loop(0, RPS // 16)
        def _(t):
            pltpu.async_copy(zbuf, acc.at[pl.ds(s * RPS + t * 16, 16)], semz)

        pltpu.sync_copy(cnt_hbm.at[w], cntv)
        cnt = cntv[pl.ds(0, 16)][lane]
        nch = (cnt + CK - 1) // CK

        @pl.loop(0, RPS // 16)
        def _(t):
            pltpu.make_async_copy(zbuf, acc.at[pl.ds(s * RPS + t * 16, 16)],
                                  semz).wait()

        plsc.subcore_barrier()

        def stage_group(gi, bi):
            pltpu.async_copy(src_hbm.at[w, gi], sgrp.at[bi], semd)
            pltpu.async_copy(dloc_hbm.at[w, gi], dgrp.at[bi], semd)

        def wait_group(gi, bi):
            pltpu.make_async_copy(src_hbm.at[w, gi], sgrp.at[bi], semd).wait()
            pltpu.make_async_copy(dloc_hbm.at[w, gi], dgrp.at[bi],
                                  semd).wait()

        def issue_gather(idxrow, r):
            pltpu.async_copy(h_hbm.at[idxrow], rows.at[r], semg[r])

        def wait_gather(idxrow, r):
            pltpu.make_async_copy(h_hbm.at[idxrow], rows.at[r],
                                  semg[r]).wait()

        def issue_scatter(bi, tk, r):
            pltpu.async_copy(rows.at[r], acc.at[dgrp.at[bi, tk]], semc[r],
                             add=True)

        def wait_scatter(bi, tk, r):
            pltpu.make_async_copy(rows.at[r], acc.at[dgrp.at[bi, tk]],
                                  semc[r]).wait()

        def slot_loop(g2, bi, t, kk):
            tk = t + kk
            j = g2 * GRP + tk
            r = kk
            r2 = (kk + 2) % 4

            @pl.when(jnp.logical_and(tk >= 2, j - 2 < nch))
            def _():
                wait_scatter(bi, tk - 2, r2)

            @pl.when(j + 2 < nch)
            def _():
                issue_gather(sgrp.at[bi, tk + 2], r2)

            @pl.when(j < nch)
            def _():
                wait_gather(sgrp.at[bi, tk], r)
                issue_scatter(bi, tk, r)

        def slot_tail(g2, bi, binext, tk):
            j = g2 * GRP + tk
            r = tk % 4
            r2 = (tk + 2) % 4

            @pl.when(j - 2 < nch)
            def _():
                wait_scatter(bi, tk - 2, r2)

            nxt = (sgrp.at[bi, tk + 2] if tk < 14
                   else sgrp.at[binext, tk - 14])

            @pl.when(j + 2 < nch)
            def _():
                issue_gather(nxt, r2)

            @pl.when(j < nch)
            def _():
                wait_gather(sgrp.at[bi, tk], r)
                issue_scatter(bi, tk, r)

        def chunks_of(g2, bi, binext):
            @pl.loop(0, 12, step=4)
            def _(t):
                slot_loop(g2, bi, t, 0)
                slot_loop(g2, bi, t, 1)
                slot_loop(g2, bi, t, 2)
                slot_loop(g2, bi, t, 3)

            for tk in (12, 13, 14, 15):
                slot_tail(g2, bi, binext, tk)

            @pl.when(g2 * GRP + 14 < nch)
            def _():
                wait_scatter(bi, 14, 2)

            @pl.when(g2 * GRP + 15 < nch)
            def _():
                wait_scatter(bi, 15, 3)

        @pl.loop(0, 12, step=3)
        def _(g):
            @pl.when(g == 0)
            def _():
                pltpu.sync_copy(src_hbm.at[w, 0], sgrp.at[0])
                pltpu.sync_copy(dloc_hbm.at[w, 0], dgrp.at[0])

                @pl.when(GRP < nch)
                def _():
                    pltpu.sync_copy(src_hbm.at[w, 1], sgrp.at[1])
                    pltpu.sync_copy(dloc_hbm.at[w, 1], dgrp.at[1])

                @pl.when(0 < nch)
                def _():
                    issue_gather(sgrp.at[0, 0], 0)

                @pl.when(1 < nch)
                def _():
                    issue_gather(sgrp.at[0, 1], 1)

            @pl.when((g + 2) * GRP < nch)
            def _():
                stage_group(g + 2, 2)

            chunks_of(g, 0, 1)

            @pl.when((g + 2) * GRP < nch)
            def _():
                wait_group(g + 2, 2)

            @pl.when((g + 3) * GRP < nch)
            def _():
                stage_group(g + 3, 0)

            chunks_of(g + 1, 1, 2)

            @pl.when((g + 3) * GRP < nch)
            def _():
                wait_group(g + 3, 0)

            @pl.when((g + 4) * GRP < nch)
            def _():
                stage_group(g + 4, 1)

            chunks_of(g + 2, 2, 0)

            @pl.when((g + 4) * GRP < nch)
            def _():
                wait_group(g + 4, 1)

        plsc.subcore_barrier()

        pltpu.sync_copy(
            acc.at[pl.ds(s * (HALF // NS), HALF // NS)],
            out_hbm.at[pl.ds(c * HALF + s * (HALF // NS), HALF // NS)],
        )

    return k(h, srcg, dlocg, counts2)


def _tc_dense(agg, w_t, b8, relu):
    blk = N_PAD // 16

    def body(a_ref, w_ref, b_ref, o_ref):
        y = jnp.dot(a_ref[...], w_ref[...], preferred_element_type=jnp.float32,
                    precision=lax.Precision.HIGHEST)
        y = y + b_ref[0:1, :]
        if relu:
            y = jnp.where(y >= 0, y, 0.01 * y)
        nrm = jnp.sqrt(jnp.sum(y * y, axis=1, keepdims=True))
        o_ref[...] = y / jnp.maximum(nrm, 1e-12)

    return pl.pallas_call(
        body,
        grid=(16,),
        in_specs=[
            pl.BlockSpec((blk, D), lambda i: (i, 0)),
            pl.BlockSpec((D, D), lambda i: (0, 0)),
            pl.BlockSpec((8, D), lambda i: (0, 0)),
        ],
        out_specs=pl.BlockSpec((blk, D), lambda i: (i, 0)),
        out_shape=jax.ShapeDtypeStruct((N_PAD, D), jnp.float32),
    )(agg, w_t, b8)


def _sc_gather_rows(h, idx):
    @functools.partial(
        pl.kernel,
        out_type=jax.ShapeDtypeStruct((G, D), jnp.float32),
        mesh=_mesh(),
        scratch_types=[
            pltpu.VMEM((G,), jnp.int32),
            pltpu.VMEM((G, D), jnp.float32),
            pltpu.SemaphoreType.DMA,
        ],
    )
    def k(h_hbm, idx_hbm, out_hbm, idx_v, rows_v, sem):
        c = lax.axis_index("c")
        s = lax.axis_index("s")

        @pl.when(jnp.logical_and(c == 0, s == 0))
        def _():
            pltpu.sync_copy(idx_hbm, idx_v)
            pltpu.async_copy(h_hbm.at[idx_v], rows_v, sem).wait()
            pltpu.sync_copy(rows_v, out_hbm)

    return k(h, idx)


def kernel(x, edge_index, batch, W1, b1, W2, b2, W3, b3):
    head = jnp.ones((1,), dtype=bool)
    changed = batch[1:] != batch[:-1]
    mask = jnp.concatenate([head, changed])
    idx = jnp.nonzero(mask, size=G, fill_value=0)[0].astype(jnp.int32)
    ishead = jnp.concatenate(
        [mask.astype(jnp.int32), jnp.zeros((N_PAD - N,), jnp.int32)])

    srcp, dlocp, srcp3, dlocp3, counts = _sc_partition(
        edge_index[0], edge_index[1], ishead)
    src4 = srcp.reshape(NW, 12, GRP, CK)
    dloc4 = dlocp.reshape(NW, 12, GRP, CK)
    src4_3 = srcp3.reshape(NW, 12, GRP, CK)
    dloc4_3 = dlocp3.reshape(NW, 12, GRP, CK)
    counts2 = counts.reshape(NW, 16)

    w1t, w2t, w3t = W1.T, W2.T, W3.T
    b1_8 = jnp.broadcast_to(b1[None, :], (8, D))
    b2_8 = jnp.broadcast_to(b2[None, :], (8, D))
    b3_8 = jnp.broadcast_to(b3[None, :], (8, D))

    a1 = _sc_aggregate(x, src4, dloc4, counts2)
    h1 = _tc_dense(a1, w1t, b1_8, relu=True)
    a2 = _sc_aggregate(h1, src4, dloc4, counts2)
    h2 = _tc_dense(a2, w2t, b2_8, relu=True)
    a3 = _sc_aggregate(h2, src4_3, dloc4_3, counts2, lane=1)
    h3 = _tc_dense(a3, w3t, b3_8, relu=False)

    return _sc_gather_rows(h3, idx)
